# Initial kernel scaffold; baseline (speedup 1.0000x reference)
import jax
import jax.numpy as jnp
from jax.experimental import pallas as pl


def kernel(keys, memory, usage):
    raise NotImplementedError("write your pallas kernel here")


# speedup vs baseline: 1.9629x; 1.9629x over previous
import jax
import jax.numpy as jnp
from jax.experimental import pallas as pl

_DIM = 128
_SIZE = 8192
_NUM_KEYS = 8
_SCALE = 5.0


def _body(keys_ref, mem_ref, usage_ref, res_ref, uout_ref):
    k = keys_ref[0]
    mem = mem_ref[0]
    u = usage_ref[0]

    kn = 1.0 / (1e-30 + jnp.sqrt(jnp.sum(k * k, axis=1, keepdims=True)))

    sim = jax.lax.dot_general(
        k, mem, (((1,), (1,)), ((), ())), preferred_element_type=jnp.float32)

    msq = jax.lax.dot_general(
        jnp.ones((1, _DIM), jnp.float32), mem * mem,
        (((1,), (1,)), ((), ())), preferred_element_type=jnp.float32)
    mn = 1.0 / (1e-30 + jnp.sqrt(msq))

    sim = sim * (kn * _SCALE) * mn

    m = jnp.max(sim, axis=1, keepdims=True)
    e = jnp.exp(sim - m)
    att = e / jnp.sum(e, axis=1, keepdims=True)

    res_ref[0] = jax.lax.dot_general(
        att, mem, (((1,), (0,)), ((), ())), preferred_element_type=jnp.float32)
    uout_ref[0] = u + jnp.sum(att, axis=0, keepdims=True)


def kernel(keys, memory, usage):
    batch = keys.shape[0]
    usage3 = usage.reshape(batch, 1, _SIZE)
    result, new_usage = pl.pallas_call(
        _body,
        grid=(batch,),
        in_specs=[
            pl.BlockSpec((1, _NUM_KEYS, _DIM), lambda b: (b, 0, 0)),
            pl.BlockSpec((1, _SIZE, _DIM), lambda b: (b, 0, 0)),
            pl.BlockSpec((1, 1, _SIZE), lambda b: (b, 0, 0)),
        ],
        out_specs=[
            pl.BlockSpec((1, _NUM_KEYS, _DIM), lambda b: (b, 0, 0)),
            pl.BlockSpec((1, 1, _SIZE), lambda b: (b, 0, 0)),
        ],
        out_shape=[
            jax.ShapeDtypeStruct((batch, _NUM_KEYS, _DIM), jnp.float32),
            jax.ShapeDtypeStruct((batch, 1, _SIZE), jnp.float32),
        ],
    )(keys, memory, usage3)
    return result, new_usage.reshape(batch, _SIZE)
